# baseline (speedup 1.0000x reference)
import functools

import jax
import jax.numpy as jnp
from jax import lax
from jax.experimental import pallas as pl
from jax.experimental.pallas import tpu as pltpu
from jax.experimental.pallas import tpu_sc as plsc

D = 768
B = 4
S = 4096
NC = 2
NS = 16
NW = NC * NS
S_PER_W = S // NW
C = 32
NCHUNK = S_PER_W // C
LANES = 16
VECS_PER_ROW = D // LANES


def _emb_kernel(x_hbm, tok_hbm, pos_hbm, out_hbm,
                pos_v, tok_v, idx_v, sem0, sem1, sem2, sem3):
    sems = (sem0, sem1, sem2, sem3)
    wid = lax.axis_index("s") * NC + lax.axis_index("c")
    s_base = wid * S_PER_W

    def chunk_body(c, carry):
        s0 = s_base + c * C
        pltpu.sync_copy(pos_hbm.at[pl.ds(s0, C)], pos_v)
        copies = []
        for b in range(B):
            pltpu.sync_copy(x_hbm.at[b, pl.ds(s0, C)], idx_v.at[b])
            cp = pltpu.async_copy(tok_hbm.at[idx_v.at[b]], tok_v.at[b],
                                  sems[b])
            copies.append(cp)
        for b in range(B):
            copies[b].wait()

            def add_row(r, _):
                for j in range(VECS_PER_ROW):
                    sl = pl.ds(j * LANES, LANES)
                    tok_v[b, r, sl] = tok_v[b, r, sl] + pos_v[r, sl]
                return 0

            lax.fori_loop(0, C, add_row, 0)
            pltpu.sync_copy(tok_v.at[b], out_hbm.at[b, pl.ds(s0, C)])
        return carry

    lax.fori_loop(0, NCHUNK, chunk_body, 0)


@jax.jit
def _emb(x, token_table, pos_table):
    mesh = plsc.VectorSubcoreMesh(core_axis_name="c", subcore_axis_name="s")
    kern = functools.partial(
        pl.kernel,
        mesh=mesh,
        out_type=jax.ShapeDtypeStruct((B, S, D), jnp.float32),
        scratch_types=[
            pltpu.VMEM((C, D), jnp.float32),
            pltpu.VMEM((B, C, D), jnp.float32),
            pltpu.VMEM((B, C), jnp.int32),
            pltpu.SemaphoreType.DMA,
            pltpu.SemaphoreType.DMA,
            pltpu.SemaphoreType.DMA,
            pltpu.SemaphoreType.DMA,
        ],
    )(_emb_kernel)
    return kern(x, token_table, pos_table)


def kernel(x, token_table, pos_table):
    return _emb(x.astype(jnp.int32), token_table, pos_table)
